# baseline (speedup 1.0000x reference)
import functools

import jax
import jax.numpy as jnp
from jax import lax
from jax.experimental import pallas as pl
from jax.experimental.pallas import tpu as pltpu


def _suppress_kernel(thresh_ref, x1_ref, y1_ref, x2_ref, y2_ref, area_ref,
                     sup_ref, acc_ref, *, n_real: int, rows: int):
    q_rows = rows // 8
    lane_f = lax.broadcasted_iota(jnp.int32, (rows, 128), 1)
    row_f = lax.broadcasted_iota(jnp.int32, (rows, 128), 0)
    pos_f = row_f * 128 + lane_f
    acc_ref[:, :] = jnp.where(pos_f >= n_real, 1.0, 0.0)
    thresh = thresh_ref[0, 0]
    lane1 = lax.broadcasted_iota(jnp.int32, (1, 128), 1)
    lane_q = lax.broadcasted_iota(jnp.int32, (q_rows, 128), 1)
    row_q = lax.broadcasted_iota(jnp.int32, (q_rows, 128), 0)

    for qa in range(8):
        posq_own = (qa * q_rows + row_q) * 128 + lane_q

        def row_body(rr, carry, qa=qa, posq_own=posq_own):
            r = qa * q_rows + rr
            x1r = x1_ref[pl.ds(r, 1), :]
            y1r = y1_ref[pl.ds(r, 1), :]
            x2r = x2_ref[pl.ds(r, 1), :]
            y2r = y2_ref[pl.ds(r, 1), :]
            arear = area_ref[pl.ds(r, 1), :]
            srow0 = acc_ref[pl.ds(r, 1), :]

            def lane_body(l, srow):
                onehot = lane1 == l
                s_i = jnp.max(jnp.where(onehot, srow, 0.0), axis=1,
                              keepdims=True)
                keep_b = s_i < 0.5
                x1_i = jnp.sum(jnp.where(onehot, x1r, 0.0), axis=1,
                               keepdims=True)
                y1_i = jnp.sum(jnp.where(onehot, y1r, 0.0), axis=1,
                               keepdims=True)
                x2_i = jnp.sum(jnp.where(onehot, x2r, 0.0), axis=1,
                               keepdims=True)
                y2_i = jnp.sum(jnp.where(onehot, y2r, 0.0), axis=1,
                               keepdims=True)
                area_i = (x2_i - x1_i) * (y2_i - y1_i)
                i = r * 128 + l
                rxx1 = jnp.maximum(x1_i, x1r)
                ryy1 = jnp.maximum(y1_i, y1r)
                rxx2 = jnp.minimum(x2_i, x2r)
                ryy2 = jnp.minimum(y2_i, y2r)
                rw = jnp.maximum(rxx2 - rxx1, 0.0)
                rh = jnp.maximum(ryy2 - ryy1, 0.0)
                rinter = rw * rh
                riou = rinter / (area_i + arear - rinter)
                rhit = (riou > thresh) & (lane1 > l) & keep_b
                srow_new = jnp.where(rhit, 1.0, srow)
                for q in range(qa, 8):
                    r0 = q * q_rows
                    x1c = x1_ref[r0:r0 + q_rows, :]
                    y1c = y1_ref[r0:r0 + q_rows, :]
                    x2c = x2_ref[r0:r0 + q_rows, :]
                    y2c = y2_ref[r0:r0 + q_rows, :]
                    areac = area_ref[r0:r0 + q_rows, :]
                    xx1 = jnp.maximum(x1_i, x1c)
                    yy1 = jnp.maximum(y1_i, y1c)
                    xx2 = jnp.minimum(x2_i, x2c)
                    yy2 = jnp.minimum(y2_i, y2c)
                    w = jnp.maximum(xx2 - xx1, 0.0)
                    h = jnp.maximum(yy2 - yy1, 0.0)
                    inter = w * h
                    iou = inter / (area_i + areac - inter)
                    hit = (iou > thresh) & keep_b
                    if q == qa:
                        hit = hit & (posq_own > i)
                    acc_ref[r0:r0 + q_rows, :] = jnp.where(
                        hit, 1.0, acc_ref[r0:r0 + q_rows, :])
                return srow_new

            srow_fin = lax.fori_loop(0, 128, lane_body, srow0)
            acc_ref[pl.ds(r, 1), :] = srow_fin
            return carry

        lax.fori_loop(0, q_rows, row_body, 0)

    sup_ref[:, :] = acc_ref[:, :]


def kernel(preds, thresh, max_proposals):
    n = preds.shape[0]
    npad = ((n + 1023) // 1024) * 1024
    rows = npad // 128

    scores = preds[:, 4]
    order = jnp.argsort(-scores)
    b = preds[order]
    coords = jnp.zeros((npad, 4), jnp.float32).at[:n].set(b[:, :4])
    x1 = coords[:, 0].reshape(rows, 128)
    y1 = coords[:, 1].reshape(rows, 128)
    x2 = coords[:, 2].reshape(rows, 128)
    y2 = coords[:, 3].reshape(rows, 128)
    areas = (x2 - x1) * (y2 - y1)
    thresh_arr = jnp.asarray(thresh, jnp.float32).reshape(1, 1)

    sup = pl.pallas_call(
        functools.partial(_suppress_kernel, n_real=n, rows=rows),
        out_shape=jax.ShapeDtypeStruct((rows, 128), jnp.float32),
        scratch_shapes=[pltpu.VMEM((rows, 128), jnp.float32)],
    )(thresh_arr, x1, y1, x2, y2, areas)

    supf = sup.reshape(-1)[:n]
    idx = jnp.arange(n, dtype=jnp.int32)
    keys = idx + supf.astype(jnp.int32) * n
    _, sel_pos = lax.top_k(-keys, 1000)
    keep1000 = order[sel_pos].astype(jnp.int32)
    sel = jnp.minimum(jnp.arange(1000), max_proposals - 1)
    keep = keep1000[sel]
    return preds[keep], keep


# speedup vs baseline: 1.0133x; 1.0133x over previous
import functools

import jax
import jax.numpy as jnp
from jax import lax
from jax.experimental import pallas as pl
from jax.experimental.pallas import tpu as pltpu


def _suppress_kernel(thresh_ref, x1_ref, y1_ref, x2_ref, y2_ref, area_ref,
                     sup_ref, acc_ref, *, n_real: int, rows: int):
    q_rows = rows // 4
    lane_f = lax.broadcasted_iota(jnp.int32, (rows, 128), 1)
    row_f = lax.broadcasted_iota(jnp.int32, (rows, 128), 0)
    pos_f = row_f * 128 + lane_f
    acc_ref[:, :] = jnp.where(pos_f >= n_real, 1.0, 0.0)
    thresh = thresh_ref[0, 0]
    lane1 = lax.broadcasted_iota(jnp.int32, (1, 128), 1)
    lane_q = lax.broadcasted_iota(jnp.int32, (q_rows, 128), 1)
    row_q = lax.broadcasted_iota(jnp.int32, (q_rows, 128), 0)

    for qa in range(4):
        posq_own = (qa * q_rows + row_q) * 128 + lane_q

        def row_body(rr, carry, qa=qa, posq_own=posq_own):
            r = qa * q_rows + rr
            x1r = x1_ref[pl.ds(r, 1), :]
            y1r = y1_ref[pl.ds(r, 1), :]
            x2r = x2_ref[pl.ds(r, 1), :]
            y2r = y2_ref[pl.ds(r, 1), :]
            arear = area_ref[pl.ds(r, 1), :]
            srow0 = acc_ref[pl.ds(r, 1), :]

            def lane_body(l, srow):
                onehot = lane1 == l
                s_i = jnp.max(jnp.where(onehot, srow, 0.0), axis=1,
                              keepdims=True)
                keep_b = s_i < 0.5
                x1_i = jnp.sum(jnp.where(onehot, x1r, 0.0), axis=1,
                               keepdims=True)
                y1_i = jnp.sum(jnp.where(onehot, y1r, 0.0), axis=1,
                               keepdims=True)
                x2_i = jnp.sum(jnp.where(onehot, x2r, 0.0), axis=1,
                               keepdims=True)
                y2_i = jnp.sum(jnp.where(onehot, y2r, 0.0), axis=1,
                               keepdims=True)
                area_i = (x2_i - x1_i) * (y2_i - y1_i)
                i = r * 128 + l
                rxx1 = jnp.maximum(x1_i, x1r)
                ryy1 = jnp.maximum(y1_i, y1r)
                rxx2 = jnp.minimum(x2_i, x2r)
                ryy2 = jnp.minimum(y2_i, y2r)
                rw = jnp.maximum(rxx2 - rxx1, 0.0)
                rh = jnp.maximum(ryy2 - ryy1, 0.0)
                rinter = rw * rh
                riou = rinter / (area_i + arear - rinter)
                rhit = (riou > thresh) & (lane1 > l) & keep_b
                srow_new = jnp.where(rhit, 1.0, srow)
                for q in range(qa, 4):
                    r0 = q * q_rows
                    x1c = x1_ref[r0:r0 + q_rows, :]
                    y1c = y1_ref[r0:r0 + q_rows, :]
                    x2c = x2_ref[r0:r0 + q_rows, :]
                    y2c = y2_ref[r0:r0 + q_rows, :]
                    areac = area_ref[r0:r0 + q_rows, :]
                    xx1 = jnp.maximum(x1_i, x1c)
                    yy1 = jnp.maximum(y1_i, y1c)
                    xx2 = jnp.minimum(x2_i, x2c)
                    yy2 = jnp.minimum(y2_i, y2c)
                    w = jnp.maximum(xx2 - xx1, 0.0)
                    h = jnp.maximum(yy2 - yy1, 0.0)
                    inter = w * h
                    iou = inter / (area_i + areac - inter)
                    hit = (iou > thresh) & keep_b
                    if q == qa:
                        hit = hit & (posq_own > i)
                    acc_ref[r0:r0 + q_rows, :] = jnp.where(
                        hit, 1.0, acc_ref[r0:r0 + q_rows, :])
                return srow_new

            srow_fin = lax.fori_loop(0, 128, lane_body, srow0)
            acc_ref[pl.ds(r, 1), :] = srow_fin
            return carry

        lax.fori_loop(0, q_rows, row_body, 0)

    sup_ref[:, :] = acc_ref[:, :]


def kernel(preds, thresh, max_proposals):
    n = preds.shape[0]
    npad = ((n + 1023) // 1024) * 1024
    rows = npad // 128

    scores = preds[:, 4]
    order = jnp.argsort(-scores)
    b = preds[order]
    coords = jnp.zeros((npad, 4), jnp.float32).at[:n].set(b[:, :4])
    x1 = coords[:, 0].reshape(rows, 128)
    y1 = coords[:, 1].reshape(rows, 128)
    x2 = coords[:, 2].reshape(rows, 128)
    y2 = coords[:, 3].reshape(rows, 128)
    areas = (x2 - x1) * (y2 - y1)
    thresh_arr = jnp.asarray(thresh, jnp.float32).reshape(1, 1)

    sup = pl.pallas_call(
        functools.partial(_suppress_kernel, n_real=n, rows=rows),
        out_shape=jax.ShapeDtypeStruct((rows, 128), jnp.float32),
        scratch_shapes=[pltpu.VMEM((rows, 128), jnp.float32)],
    )(thresh_arr, x1, y1, x2, y2, areas)

    supf = sup.reshape(-1)[:n]
    idx = jnp.arange(n, dtype=jnp.int32)
    keys = idx + supf.astype(jnp.int32) * n
    _, sel_pos = lax.top_k(-keys, 1000)
    keep1000 = order[sel_pos].astype(jnp.int32)
    sel = jnp.minimum(jnp.arange(1000), max_proposals - 1)
    keep = keep1000[sel]
    return preds[keep], keep
